# baseline (speedup 1.0000x reference)
import functools

import jax
import jax.numpy as jnp
from jax import lax
from jax.experimental import pallas as pl
from jax.experimental.pallas import tpu as pltpu
from jax.experimental.pallas import tpu_sc as plsc

N_NODES = 10000
D = 128
DH = 64
NC = 2
NS = 16
CHUNK = 200
CHUNKS_PER_T = 100
WCH = 20
NWIN = CHUNKS_PER_T // WCH
NBUF = 4
ROWS_PER_TILE = N_NODES // NS
DST_BASE = NS * CHUNKS_PER_T

_sc_mesh = plsc.VectorSubcoreMesh(core_axis_name="c", subcore_axis_name="s")


@functools.partial(
    pl.kernel,
    out_type=(jax.ShapeDtypeStruct((N_NODES, D), jnp.float32),
              jax.ShapeDtypeStruct((NC, N_NODES, DH), jnp.float32)),
    mesh=_sc_mesh,
    scratch_types=[
        pltpu.VMEM_SHARED((N_NODES, DH), jnp.float32),
        pltpu.VMEM((CHUNKS_PER_T, CHUNK), jnp.int32),
        pltpu.VMEM((WCH, CHUNK), jnp.int32),
        [pltpu.VMEM((CHUNK, DH), jnp.float32) for _ in range(NBUF)],
        [pltpu.SemaphoreType.DMA for _ in range(NBUF)],
    ],
    compiler_params=pltpu.CompilerParams(use_tc_tiling_on_sc=False),
)
def _sc_aggregate(x_hbm, ei_hbm, zeros_hbm, agg_out, xs_hbm,
                  agg_sh, src_v, dst_v, rows, sems):
    c = lax.axis_index("c")
    s = lax.axis_index("s")
    xsc = xs_hbm.at[c]
    stripe = pl.ds(s * ROWS_PER_TILE, ROWS_PER_TILE)

    pltpu.sync_copy(x_hbm.at[stripe, pl.ds(c * DH, DH)], xsc.at[stripe])
    pltpu.sync_copy(zeros_hbm.at[stripe], agg_sh.at[stripe])
    pltpu.sync_copy(ei_hbm.at[pl.ds(s * CHUNKS_PER_T, CHUNKS_PER_T)], src_v)
    plsc.subcore_barrier()

    def wait(buf, sem):
        pltpu.make_async_copy(xsc.at[src_v.at[0]], buf, sem).wait()

    for k in range(NBUF):
        pltpu.async_copy(xsc.at[src_v.at[k]], rows[k], sems[k])

    def window(win, carry):
        pltpu.sync_copy(
            ei_hbm.at[pl.ds(DST_BASE + s * CHUNKS_PER_T + win * WCH, WCH)],
            dst_v)

        def quad(q, c2):
            j = win * WCH + NBUF * q
            for k in range(NBUF):
                wait(rows[k], sems[k])
                pltpu.sync_copy(rows[k], agg_sh.at[dst_v.at[NBUF * q + k]],
                                add=True)

                @pl.when(j + k + NBUF < CHUNKS_PER_T)
                def _():
                    pltpu.async_copy(xsc.at[src_v.at[j + k + NBUF]],
                                     rows[k], sems[k])

            return c2

        lax.fori_loop(0, WCH // NBUF, quad, carry, unroll=False)
        return carry

    lax.fori_loop(0, NWIN, window, 0, unroll=False)

    plsc.subcore_barrier()
    pltpu.sync_copy(agg_sh.at[stripe],
                    agg_out.at[stripe, pl.ds(c * DH, DH)])


def _tc_body(x_ref, a_ref, wxt_ref, wat_ref, b_ref, o_ref):
    acc = jnp.dot(x_ref[...], wxt_ref[...],
                  preferred_element_type=jnp.float32)
    acc = acc + jnp.dot(a_ref[...], wat_ref[...],
                        preferred_element_type=jnp.float32)
    acc = acc + b_ref[...]
    acc = jnp.maximum(acc, 0.0)
    ss = jnp.sum(acc * acc, axis=1, keepdims=True)
    o_ref[...] = acc * lax.rsqrt(jnp.maximum(ss, 1e-24))


BN = 2000


def _tc_dense(x, agg, wxt, wat, b2):
    return pl.pallas_call(
        _tc_body,
        grid=(N_NODES // BN,),
        in_specs=[
            pl.BlockSpec((BN, D), lambda i: (i, 0)),
            pl.BlockSpec((BN, D), lambda i: (i, 0)),
            pl.BlockSpec((D, D), lambda i: (0, 0)),
            pl.BlockSpec((D, D), lambda i: (0, 0)),
            pl.BlockSpec((1, D), lambda i: (0, 0)),
        ],
        out_specs=pl.BlockSpec((BN, D), lambda i: (i, 0)),
        out_shape=jax.ShapeDtypeStruct((N_NODES, D), jnp.float32),
    )(x, agg, wxt, wat, b2)


def kernel(x, edge_index, W, b):
    x = x.astype(jnp.float32)
    ei2 = edge_index.astype(jnp.int32).reshape(2 * NS * CHUNKS_PER_T, CHUNK)
    zeros = jnp.zeros((N_NODES, DH), jnp.float32)

    agg, _ = _sc_aggregate(x, ei2, zeros)

    wxt = W[:, :D].T
    wat = W[:, D:].T
    b2 = b.reshape(1, D)
    return _tc_dense(x, agg, wxt, wat, b2)


# speedup vs baseline: 2.1611x; 2.1611x over previous
import functools

import jax
import jax.numpy as jnp
from jax import lax
from jax.experimental import pallas as pl
from jax.experimental.pallas import tpu as pltpu
from jax.experimental.pallas import tpu_sc as plsc

N_NODES = 10000
D = 128
DH = 64
NC = 2
NS = 16
CHUNK = 200
CHUNKS_PER_T = 100
WCH = 20
NWIN = CHUNKS_PER_T // WCH
NBUF = 4
ROWS_PER_TILE = N_NODES // NS
DST_BASE = NS * CHUNKS_PER_T

_sc_mesh = plsc.VectorSubcoreMesh(core_axis_name="c", subcore_axis_name="s")


@functools.partial(
    pl.kernel,
    out_type=(jax.ShapeDtypeStruct((N_NODES, D), jnp.float32),
              jax.ShapeDtypeStruct((NC, N_NODES, DH), jnp.float32)),
    mesh=_sc_mesh,
    scratch_types=[
        pltpu.VMEM_SHARED((N_NODES, DH), jnp.float32),
        pltpu.VMEM((CHUNKS_PER_T, CHUNK), jnp.int32),
        pltpu.VMEM((WCH, CHUNK), jnp.int32),
        [pltpu.VMEM((CHUNK, DH), jnp.float32) for _ in range(NBUF)],
        [pltpu.SemaphoreType.DMA for _ in range(NBUF)],
    ],
    compiler_params=pltpu.CompilerParams(use_tc_tiling_on_sc=False),
)
def _sc_aggregate(x_hbm, ei_hbm, zeros_hbm, agg_out, xs_hbm,
                  agg_sh, src_v, dst_v, rows, sems):
    c = lax.axis_index("c")
    s = lax.axis_index("s")
    xsc = xs_hbm.at[c]
    stripe = pl.ds(s * ROWS_PER_TILE, ROWS_PER_TILE)

    pltpu.sync_copy(x_hbm.at[stripe, pl.ds(c * DH, DH)], agg_sh.at[stripe])
    pltpu.sync_copy(agg_sh.at[stripe], xsc.at[stripe])
    pltpu.sync_copy(zeros_hbm.at[stripe], agg_sh.at[stripe])
    pltpu.sync_copy(ei_hbm.at[pl.ds(s * CHUNKS_PER_T, CHUNKS_PER_T)], src_v)
    plsc.subcore_barrier()

    def wait(buf, sem):
        pltpu.make_async_copy(xsc.at[src_v.at[0]], buf, sem).wait()

    for k in range(NBUF):
        pltpu.async_copy(xsc.at[src_v.at[k]], rows[k], sems[k])

    def window(win, carry):
        pltpu.sync_copy(
            ei_hbm.at[pl.ds(DST_BASE + s * CHUNKS_PER_T + win * WCH, WCH)],
            dst_v)

        def quad(q, c2):
            j = win * WCH + NBUF * q
            for k in range(NBUF):
                wait(rows[k], sems[k])
                pltpu.sync_copy(rows[k], agg_sh.at[dst_v.at[NBUF * q + k]],
                                add=True)

                @pl.when(j + k + NBUF < CHUNKS_PER_T)
                def _():
                    pltpu.async_copy(xsc.at[src_v.at[j + k + NBUF]],
                                     rows[k], sems[k])

            return c2

        lax.fori_loop(0, WCH // NBUF, quad, carry, unroll=False)
        return carry

    lax.fori_loop(0, NWIN, window, 0, unroll=False)

    plsc.subcore_barrier()
    pltpu.sync_copy(agg_sh.at[stripe],
                    agg_out.at[stripe, pl.ds(c * DH, DH)])


def _tc_body(x_ref, a_ref, wxt_ref, wat_ref, b_ref, o_ref):
    acc = jnp.dot(x_ref[...], wxt_ref[...],
                  preferred_element_type=jnp.float32)
    acc = acc + jnp.dot(a_ref[...], wat_ref[...],
                        preferred_element_type=jnp.float32)
    acc = acc + b_ref[...]
    acc = jnp.maximum(acc, 0.0)
    ss = jnp.sum(acc * acc, axis=1, keepdims=True)
    o_ref[...] = acc * lax.rsqrt(jnp.maximum(ss, 1e-24))


BN = 2000


def _tc_dense(x, agg, wxt, wat, b2):
    return pl.pallas_call(
        _tc_body,
        grid=(N_NODES // BN,),
        in_specs=[
            pl.BlockSpec((BN, D), lambda i: (i, 0)),
            pl.BlockSpec((BN, D), lambda i: (i, 0)),
            pl.BlockSpec((D, D), lambda i: (0, 0)),
            pl.BlockSpec((D, D), lambda i: (0, 0)),
            pl.BlockSpec((1, D), lambda i: (0, 0)),
        ],
        out_specs=pl.BlockSpec((BN, D), lambda i: (i, 0)),
        out_shape=jax.ShapeDtypeStruct((N_NODES, D), jnp.float32),
    )(x, agg, wxt, wat, b2)


def kernel(x, edge_index, W, b):
    x = x.astype(jnp.float32)
    ei2 = edge_index.astype(jnp.int32).reshape(2 * NS * CHUNKS_PER_T, CHUNK)
    zeros = jnp.zeros((N_NODES, DH), jnp.float32)

    agg, _ = _sc_aggregate(x, ei2, zeros)

    wxt = W[:, :D].T
    wat = W[:, D:].T
    b2 = b.reshape(1, D)
    return _tc_dense(x, agg, wxt, wat, b2)
